# Initial kernel scaffold; baseline (speedup 1.0000x reference)
import jax
import jax.numpy as jnp
from jax.experimental import pallas as pl


def kernel(x, edge_index, batch, W1, gamma1, beta1, W2, gamma2, beta2, eps):
    raise NotImplementedError("write your pallas kernel here")


# speedup vs baseline: 7.2771x; 7.2771x over previous
import functools

import jax
import jax.numpy as jnp
from jax import lax
from jax.experimental import pallas as pl
from jax.experimental.pallas import tpu as pltpu
from jax.experimental.pallas import tpu_sc as plsc

_N_NODES = 10000
_D = 128
_N_EDGES = 320000
_BN_EPS = 1e-5

_NC = 2
_NS = 16
_NW = _NC * _NS
_EPW = _N_EDGES // _NW
_K = 80
_NCHUNK = _EPW // _K
_NPAD = 10240
_RPT = _NPAD // _NS
_ZROWS = 32


@functools.partial(
    pl.kernel,
    out_type=jax.ShapeDtypeStruct((_NC, _NPAD, _D), jnp.float32),
    mesh=plsc.VectorSubcoreMesh(core_axis_name="c", subcore_axis_name="s"),
    scratch_types=[
        pltpu.VMEM((_NCHUNK, _K), jnp.int32),
        pltpu.VMEM((_NCHUNK, _K), jnp.int32),
        pltpu.VMEM((_K, _D), jnp.float32),
        pltpu.VMEM((_ZROWS, _D), jnp.float32),
        pltpu.VMEM_SHARED((_NPAD, _D), jnp.float32),
        pltpu.SemaphoreType.DMA,
    ],
)
def _sc_segment_sum(x_hbm, src_hbm, dst_hbm, out_hbm,
                    src_v, dst_v, rows_v, zbuf, agg_sh, sem):
    cid = lax.axis_index("c")
    sid = lax.axis_index("s")
    wid = cid * _NS + sid

    zeros16 = jnp.zeros((16,), jnp.float32)

    @pl.loop(0, _ZROWS)
    def _(i):
        @pl.loop(0, _D // 16)
        def _(j):
            zbuf.at[i, pl.ds(j * 16, 16)][...] = zeros16

    @pl.loop(0, _RPT // _ZROWS)
    def _(r):
        pltpu.sync_copy(zbuf, agg_sh.at[pl.ds(sid * _RPT + r * _ZROWS, _ZROWS)])

    pltpu.sync_copy(src_hbm.at[wid], src_v)
    pltpu.sync_copy(dst_hbm.at[wid], dst_v)

    plsc.subcore_barrier()

    @pl.loop(0, _NCHUNK)
    def _(i):
        pltpu.async_copy(x_hbm.at[src_v.at[i]], rows_v, sem).wait()
        pltpu.sync_copy(rows_v, agg_sh.at[dst_v.at[i]], add=True)

    plsc.subcore_barrier()
    pltpu.sync_copy(agg_sh.at[pl.ds(sid * _RPT, _RPT)],
                    out_hbm.at[cid].at[pl.ds(sid * _RPT, _RPT)])


def _mlp_body(x_ref, agg_ref, w1_ref, g1_ref, b1_ref, w2_ref, g2_ref, b2_ref,
              eps_ref, o_ref):
    h = ((1.0 + eps_ref[0, 0]) * x_ref[...]
         + agg_ref[0, :_N_NODES, :] + agg_ref[1, :_N_NODES, :])
    dn = (((1,), (1,)), ((), ()))
    h = lax.dot_general(h, w1_ref[...], dn,
                        preferred_element_type=jnp.float32,
                        precision=lax.Precision.HIGHEST)
    mu = jnp.mean(h, axis=0, keepdims=True)
    var = jnp.mean((h - mu) ** 2, axis=0, keepdims=True)
    h = g1_ref[...] * (h - mu) * lax.rsqrt(var + _BN_EPS) + b1_ref[...]
    h = jnp.maximum(h, 0.0)
    h = lax.dot_general(h, w2_ref[...], dn,
                        preferred_element_type=jnp.float32,
                        precision=lax.Precision.HIGHEST)
    mu = jnp.mean(h, axis=0, keepdims=True)
    var = jnp.mean((h - mu) ** 2, axis=0, keepdims=True)
    h = g2_ref[...] * (h - mu) * lax.rsqrt(var + _BN_EPS) + b2_ref[...]
    o_ref[...] = jnp.maximum(h, 0.0)


@jax.jit
def kernel(x, edge_index, batch, W1, gamma1, beta1, W2, gamma2, beta2, eps):
    del batch
    ei = edge_index.astype(jnp.int32)
    src = ei[0].reshape(_NW, _NCHUNK, _K)
    dst = ei[1].reshape(_NW, _NCHUNK, _K)
    agg = _sc_segment_sum(x, src, dst)
    return pl.pallas_call(
        _mlp_body,
        out_shape=jax.ShapeDtypeStruct((_N_NODES, _D), jnp.float32),
    )(x, agg, W1, gamma1.reshape(1, _D), beta1.reshape(1, _D),
      W2, gamma2.reshape(1, _D), beta2.reshape(1, _D),
      eps.reshape(1, 1).astype(jnp.float32))


# speedup vs baseline: 8.5711x; 1.1778x over previous
import functools

import jax
import jax.numpy as jnp
from jax import lax
from jax.experimental import pallas as pl
from jax.experimental.pallas import tpu as pltpu
from jax.experimental.pallas import tpu_sc as plsc

_N_NODES = 10000
_D = 128
_N_EDGES = 320000
_BN_EPS = 1e-5

_NC = 2
_NS = 16
_NW = _NC * _NS
_EPW = _N_EDGES // _NW
_K = 80
_NCHUNK = _EPW // _K
_NPAD = 10240
_RPT = _NPAD // _NS


@functools.partial(
    pl.kernel,
    out_type=jax.ShapeDtypeStruct((_NC, _NPAD, _D), jnp.float32),
    mesh=plsc.VectorSubcoreMesh(core_axis_name="c", subcore_axis_name="s"),
    scratch_types=[
        pltpu.VMEM((_K, _D), jnp.float32),
        pltpu.VMEM((_K, _D), jnp.float32),
        pltpu.VMEM((2, _K), jnp.int32),
        pltpu.VMEM((2, _K), jnp.int32),
        pltpu.VMEM_SHARED((_NPAD, _D), jnp.float32),
        pltpu.SemaphoreType.DMA,
        pltpu.SemaphoreType.DMA,
        pltpu.SemaphoreType.DMA,
        pltpu.SemaphoreType.DMA,
    ],
)
def _sc_segment_sum(x_hbm, e_hbm, out_hbm,
                    rows0, rows1, idx0, idx1, agg_sh,
                    sem0, sem1, semi0, semi1):
    cid = lax.axis_index("c")
    sid = lax.axis_index("s")
    wid = cid * _NS + sid

    rows = (rows0, rows1)
    idx = (idx0, idx1)
    gsem = (sem0, sem1)
    isem = (semi0, semi1)

    zeros16 = jnp.zeros((16,), jnp.float32)

    @pl.loop(0, _K)
    def _(i):
        @pl.loop(0, _D // 16)
        def _(j):
            rows0.at[i, pl.ds(j * 16, 16)][...] = zeros16

    @pl.loop(0, _RPT // _K)
    def _(r):
        pltpu.sync_copy(rows0, agg_sh.at[pl.ds(sid * _RPT + r * _K, _K)])

    plsc.subcore_barrier()

    pltpu.sync_copy(e_hbm.at[wid, 0], idx0)
    pltpu.async_copy(x_hbm.at[idx0.at[0]], rows0, sem0)
    pltpu.async_copy(e_hbm.at[wid, 1], idx1, semi1)

    def _step(i, p, issue_next_gather, issue_next_idx):
        q = 1 - p
        pltpu.make_async_copy(x_hbm.at[idx[p].at[0]], rows[p], gsem[p]).wait()
        if issue_next_gather:
            pltpu.make_async_copy(e_hbm.at[wid, i + 1], idx[q], isem[q]).wait()
            pltpu.async_copy(x_hbm.at[idx[q].at[0]], rows[q], gsem[q])
        pltpu.sync_copy(rows[p], agg_sh.at[idx[p].at[1]], add=True)
        if issue_next_idx:
            pltpu.async_copy(e_hbm.at[wid, i + 2], idx[p], isem[p])

    @pl.loop(0, (_NCHUNK - 3) // 2)
    def _(g):
        _step(2 * g, 0, True, True)
        _step(2 * g + 1, 1, True, True)

    _step(_NCHUNK - 3, 0, True, True)
    _step(_NCHUNK - 2, 1, True, False)
    _step(_NCHUNK - 1, 0, False, False)

    plsc.subcore_barrier()
    pltpu.sync_copy(agg_sh.at[pl.ds(sid * _RPT, _RPT)],
                    out_hbm.at[cid].at[pl.ds(sid * _RPT, _RPT)])


def _mlp_body(x_ref, agg_ref, w1_ref, g1_ref, b1_ref, w2_ref, g2_ref, b2_ref,
              eps_ref, o_ref):
    h = ((1.0 + eps_ref[0, 0]) * x_ref[...]
         + agg_ref[0, :_N_NODES, :] + agg_ref[1, :_N_NODES, :])
    dn = (((1,), (1,)), ((), ()))
    h = lax.dot_general(h, w1_ref[...], dn,
                        preferred_element_type=jnp.float32,
                        precision=lax.Precision.HIGHEST)
    mu = jnp.mean(h, axis=0, keepdims=True)
    var = jnp.mean((h - mu) ** 2, axis=0, keepdims=True)
    h = g1_ref[...] * (h - mu) * lax.rsqrt(var + _BN_EPS) + b1_ref[...]
    h = jnp.maximum(h, 0.0)
    h = lax.dot_general(h, w2_ref[...], dn,
                        preferred_element_type=jnp.float32,
                        precision=lax.Precision.HIGHEST)
    mu = jnp.mean(h, axis=0, keepdims=True)
    var = jnp.mean((h - mu) ** 2, axis=0, keepdims=True)
    h = g2_ref[...] * (h - mu) * lax.rsqrt(var + _BN_EPS) + b2_ref[...]
    o_ref[...] = jnp.maximum(h, 0.0)


@jax.jit
def kernel(x, edge_index, batch, W1, gamma1, beta1, W2, gamma2, beta2, eps):
    del batch
    ei = edge_index.astype(jnp.int32)
    e = jnp.stack([ei[0].reshape(_NW, _NCHUNK, _K),
                   ei[1].reshape(_NW, _NCHUNK, _K)], axis=2)
    agg = _sc_segment_sum(x, e)
    return pl.pallas_call(
        _mlp_body,
        out_shape=jax.ShapeDtypeStruct((_N_NODES, _D), jnp.float32),
    )(x, agg, W1, gamma1.reshape(1, _D), beta1.reshape(1, _D),
      W2, gamma2.reshape(1, _D), beta2.reshape(1, _D),
      eps.reshape(1, 1).astype(jnp.float32))
